# baseline (speedup 1.0000x reference)
import functools

import jax
import jax.numpy as jnp
from jax import lax
from jax.experimental import pallas as pl
from jax.experimental.pallas import tpu as pltpu
from jax.experimental.pallas import tpu_sc as plsc

B, F, V, D = 4096, 26, 100000, 16
FD = F * D
HID = (256, 128, 64)

NC, NS = 2, 16
NW = NC * NS
PPT = FD // NW
BPT = B // NW


def _sc_gather(tableT, fmseg, inputsT):
    mesh = plsc.VectorSubcoreMesh(core_axis_name="c", subcore_axis_name="s")

    @functools.partial(
        pl.kernel,
        mesh=mesh,
        compiler_params=pltpu.CompilerParams(use_tc_tiling_on_sc=True,
                                             needs_layout_passes=False),
        out_type=[
            jax.ShapeDtypeStruct((FD, B), jnp.float32),
            jax.ShapeDtypeStruct((B,), jnp.float32),
        ],
        scratch_types=[
            pltpu.VMEM((V,), jnp.float32),
            pltpu.VMEM((2, B), jnp.int32),
            pltpu.VMEM((2, B), jnp.float32),
            pltpu.VMEM((BPT,), jnp.int32),
            pltpu.VMEM((BPT,), jnp.float32),
            pltpu.SemaphoreType.DMA,
            pltpu.SemaphoreType.DMA,
        ],
    )
    def gather_kernel(tableT_hbm, fmseg_hbm, inputsT_hbm,
                      embT_out, fm_out,
                      plane_v, idx_v, out_v, fidx_v, fm1_v, sem_p, sem_o):
        wid = lax.axis_index("s") * NC + lax.axis_index("c")
        p0 = wid * PPT
        f0 = p0 // D
        f1 = (p0 + PPT - 1) // D

        pltpu.sync_copy(inputsT_hbm.at[f0], idx_v.at[0])
        pltpu.sync_copy(inputsT_hbm.at[f1], idx_v.at[1])

        out_cp = [None, None]
        for i in range(PPT):
            p = p0 + i
            f = p // D
            d = p % D
            pltpu.async_copy(tableT_hbm.at[f, d], plane_v, sem_p).wait()
            if out_cp[i % 2] is not None:
                out_cp[i % 2].wait()
            sel = jnp.where(f == f0, 0, 1)

            def body(j, _):
                for u in range(4):
                    jj = j * 4 + u
                    idx16 = idx_v[sel, pl.ds(jj * 16, 16)]
                    out_v[i % 2, pl.ds(jj * 16, 16)] = plsc.load_gather(
                        plane_v, [idx16])
                return ()

            lax.fori_loop(0, B // 64, body, ())
            out_cp[i % 2] = pltpu.async_copy(
                out_v.at[i % 2], embT_out.at[p], sem_o)
        for c in out_cp:
            if c is not None:
                c.wait()

        pltpu.sync_copy(fmseg_hbm, plane_v)
        for j in range(BPT // 16):
            fm1_v[pl.ds(j * 16, 16)] = jnp.zeros((16,), jnp.float32)
        for f in range(F):
            pltpu.sync_copy(inputsT_hbm.at[f, pl.ds(wid * BPT, BPT)], fidx_v)
            for j in range(BPT // 16):
                sl = pl.ds(j * 16, 16)
                fm1_v[sl] = fm1_v[sl] + plsc.load_gather(plane_v, [fidx_v[sl]])
        pltpu.sync_copy(fm1_v, fm_out.at[pl.ds(wid * BPT, BPT)])

    return gather_kernel(tableT, fmseg, inputsT)


def _mlp_body(x_ref, fm_ref, w0_ref, b0_ref, w1_ref, b1_ref, w2_ref, b2_ref,
              dw_ref, db_ref, out_ref):
    x = x_ref[...]
    first = fm_ref[...]
    row = lax.broadcasted_iota(jnp.int32, (FD, D), 0)
    col = lax.broadcasted_iota(jnp.int32, (FD, D), 1)
    m = (row % D == col).astype(jnp.float32)
    cdim = (((0,), (0,)), ((), ()))
    s = lax.dot_general(m, x, cdim, preferred_element_type=jnp.float32)
    second = 0.5 * (jnp.sum(s * s, axis=0, keepdims=True)
                    - jnp.sum(x * x, axis=0, keepdims=True))
    h = jnp.maximum(
        lax.dot_general(w0_ref[...], x, cdim,
                        preferred_element_type=jnp.float32) + b0_ref[...], 0.0)
    h = jnp.maximum(
        lax.dot_general(w1_ref[...], h, cdim,
                        preferred_element_type=jnp.float32) + b1_ref[...], 0.0)
    h = jnp.maximum(
        lax.dot_general(w2_ref[...], h, cdim,
                        preferred_element_type=jnp.float32) + b2_ref[...], 0.0)
    deep = lax.dot_general(dw_ref[...], h, cdim,
                           preferred_element_type=jnp.float32) + db_ref[...]
    out_ref[...] = jax.nn.sigmoid(first + second + deep)


def _tc_mlp(embT, fm_first, W0, b0, W1, b1, W2, b2, dense_W, dense_b):
    BB = 1024
    grid = (B // BB,)
    const = lambda shape: pl.BlockSpec(shape, lambda i: (0, 0))
    return pl.pallas_call(
        _mlp_body,
        grid=grid,
        in_specs=[
            pl.BlockSpec((FD, BB), lambda i: (0, i)),
            pl.BlockSpec((1, BB), lambda i: (0, i)),
            const((FD, HID[0])),
            const((HID[0], 1)),
            const((HID[0], HID[1])),
            const((HID[1], 1)),
            const((HID[1], HID[2])),
            const((HID[2], 1)),
            const((HID[2], 1)),
            const((1, 1)),
        ],
        out_specs=pl.BlockSpec((1, BB), lambda i: (0, i)),
        out_shape=jax.ShapeDtypeStruct((1, B), jnp.float32),
    )(embT, fm_first.reshape(1, B), W0, b0.reshape(-1, 1), W1,
      b1.reshape(-1, 1), W2, b2.reshape(-1, 1), dense_W,
      dense_b.reshape(1, 1))


def kernel(inputs, emb_tables, fm_w, W0, b0, W1, b1, W2, b2, dense_W, dense_b):
    tableT = emb_tables.transpose(0, 2, 1)
    fmseg = fm_w[V:2 * V, 0]
    inputsT = inputs.T
    embT, fm_first = _sc_gather(tableT, fmseg, inputsT)
    out = _tc_mlp(embT, fm_first, W0, b0, W1, b1, W2, b2, dense_W, dense_b)
    return out.reshape(B, 1)


# speedup vs baseline: 1.0492x; 1.0492x over previous
import functools

import jax
import jax.numpy as jnp
from jax import lax
from jax.experimental import pallas as pl
from jax.experimental.pallas import tpu as pltpu
from jax.experimental.pallas import tpu_sc as plsc

B, F, V, D = 4096, 26, 100000, 16
FD = F * D
HID = (256, 128, 64)

NC, NS = 2, 16
NW = NC * NS
PPT = FD // NW
BPT = B // NW


def _sc_gather(tableT, fmseg, inputsT, fidx3):
    mesh = plsc.VectorSubcoreMesh(core_axis_name="c", subcore_axis_name="s")

    @functools.partial(
        pl.kernel,
        mesh=mesh,
        compiler_params=pltpu.CompilerParams(use_tc_tiling_on_sc=True,
                                             needs_layout_passes=False),
        out_type=[
            jax.ShapeDtypeStruct((FD, B), jnp.float32),
            jax.ShapeDtypeStruct((B,), jnp.float32),
        ],
        scratch_types=[
            pltpu.VMEM((V,), jnp.float32),
            pltpu.VMEM((2, B), jnp.int32),
            pltpu.VMEM((2, B), jnp.float32),
            pltpu.VMEM((F, BPT), jnp.int32),
            pltpu.VMEM((BPT,), jnp.float32),
            pltpu.SemaphoreType.DMA,
            pltpu.SemaphoreType.DMA,
        ],
    )
    def gather_kernel(tableT_hbm, fmseg_hbm, inputsT_hbm, fidx_hbm,
                      embT_out, fm_out,
                      plane_v, idx_v, out_v, fidx_v, fm1_v, sem_p, sem_o):
        wid = lax.axis_index("s") * NC + lax.axis_index("c")
        p0 = wid * PPT
        f0 = p0 // D
        f1 = (p0 + PPT - 1) // D

        pltpu.sync_copy(inputsT_hbm.at[f0], idx_v.at[0])
        pltpu.sync_copy(inputsT_hbm.at[f1], idx_v.at[1])

        out_cp = [None, None]
        for i in range(PPT):
            p = p0 + i
            f = p // D
            d = p % D
            pltpu.async_copy(tableT_hbm.at[f, d], plane_v, sem_p).wait()
            if out_cp[i % 2] is not None:
                out_cp[i % 2].wait()
            sel = jnp.where(f == f0, 0, 1)

            def body(j, _):
                for u in range(4):
                    jj = j * 4 + u
                    idx16 = idx_v[sel, pl.ds(jj * 16, 16)]
                    out_v[i % 2, pl.ds(jj * 16, 16)] = plsc.load_gather(
                        plane_v, [idx16])
                return ()

            lax.fori_loop(0, B // 64, body, ())
            out_cp[i % 2] = pltpu.async_copy(
                out_v.at[i % 2], embT_out.at[p], sem_o)
        for c in out_cp:
            if c is not None:
                c.wait()

        pltpu.sync_copy(fmseg_hbm, plane_v)
        pltpu.sync_copy(fidx_hbm.at[wid], fidx_v)

        def fm_body(j, _):
            acc = jnp.zeros((16,), jnp.float32)
            for f in range(F):
                idx16 = fidx_v[f, pl.ds(j * 16, 16)]
                acc = acc + plsc.load_gather(plane_v, [idx16])
            fm1_v[pl.ds(j * 16, 16)] = acc
            return ()

        lax.fori_loop(0, BPT // 16, fm_body, ())
        pltpu.sync_copy(fm1_v, fm_out.at[pl.ds(wid * BPT, BPT)])

    return gather_kernel(tableT, fmseg, inputsT, fidx3)


def _mlp_body(x_ref, fm_ref, w0_ref, b0_ref, w1_ref, b1_ref, w2_ref, b2_ref,
              dw_ref, db_ref, out_ref):
    x = x_ref[...]
    first = fm_ref[...]
    row = lax.broadcasted_iota(jnp.int32, (FD, D), 0)
    col = lax.broadcasted_iota(jnp.int32, (FD, D), 1)
    m = (row % D == col).astype(jnp.float32)
    cdim = (((0,), (0,)), ((), ()))
    s = lax.dot_general(m, x, cdim, preferred_element_type=jnp.float32)
    second = 0.5 * (jnp.sum(s * s, axis=0, keepdims=True)
                    - jnp.sum(x * x, axis=0, keepdims=True))
    h = jnp.maximum(
        lax.dot_general(w0_ref[...], x, cdim,
                        preferred_element_type=jnp.float32) + b0_ref[...], 0.0)
    h = jnp.maximum(
        lax.dot_general(w1_ref[...], h, cdim,
                        preferred_element_type=jnp.float32) + b1_ref[...], 0.0)
    h = jnp.maximum(
        lax.dot_general(w2_ref[...], h, cdim,
                        preferred_element_type=jnp.float32) + b2_ref[...], 0.0)
    deep = lax.dot_general(dw_ref[...], h, cdim,
                           preferred_element_type=jnp.float32) + db_ref[...]
    out_ref[...] = jax.nn.sigmoid(first + second + deep)


def _tc_mlp(embT, fm_first, W0, b0, W1, b1, W2, b2, dense_W, dense_b):
    BB = 512
    grid = (B // BB,)
    const = lambda shape: pl.BlockSpec(shape, lambda i: (0, 0))
    return pl.pallas_call(
        _mlp_body,
        grid=grid,
        in_specs=[
            pl.BlockSpec((FD, BB), lambda i: (0, i)),
            pl.BlockSpec((1, BB), lambda i: (0, i)),
            const((FD, HID[0])),
            const((HID[0], 1)),
            const((HID[0], HID[1])),
            const((HID[1], 1)),
            const((HID[1], HID[2])),
            const((HID[2], 1)),
            const((HID[2], 1)),
            const((1, 1)),
        ],
        out_specs=pl.BlockSpec((1, BB), lambda i: (0, i)),
        out_shape=jax.ShapeDtypeStruct((1, B), jnp.float32),
    )(embT, fm_first.reshape(1, B), W0, b0.reshape(-1, 1), W1,
      b1.reshape(-1, 1), W2, b2.reshape(-1, 1), dense_W,
      dense_b.reshape(1, 1))


def kernel(inputs, emb_tables, fm_w, W0, b0, W1, b1, W2, b2, dense_W, dense_b):
    tableT = emb_tables.transpose(0, 2, 1)
    fmseg = fm_w[V:2 * V, 0]
    inputsT = inputs.T
    fidx3 = inputsT.reshape(F, NW, BPT).transpose(1, 0, 2)
    embT, fm_first = _sc_gather(tableT, fmseg, inputsT, fidx3)
    out = _tc_mlp(embT, fm_first, W0, b0, W1, b1, W2, b2, dense_W, dense_b)
    return out.reshape(B, 1)


# speedup vs baseline: 1.0744x; 1.0240x over previous
import functools

import jax
import jax.numpy as jnp
from jax import lax
from jax.experimental import pallas as pl
from jax.experimental.pallas import tpu as pltpu
from jax.experimental.pallas import tpu_sc as plsc

B, F, V, D = 4096, 26, 100000, 16
FD = F * D
HID = (256, 128, 64)

NC, NS = 2, 16
NW = NC * NS
PPT = FD // NW
BPT = B // NW


def _sc_gather(tableT, fmseg, inputsT, fidx3):
    mesh = plsc.VectorSubcoreMesh(core_axis_name="c", subcore_axis_name="s")

    @functools.partial(
        pl.kernel,
        mesh=mesh,
        compiler_params=pltpu.CompilerParams(use_tc_tiling_on_sc=True,
                                             needs_layout_passes=False),
        out_type=[
            jax.ShapeDtypeStruct((FD, B), jnp.float32),
            jax.ShapeDtypeStruct((B,), jnp.float32),
        ],
        scratch_types=[
            pltpu.VMEM((V,), jnp.float32),
            pltpu.VMEM((2, B), jnp.int32),
            pltpu.VMEM((2, B), jnp.float32),
            pltpu.VMEM((F, BPT), jnp.int32),
            pltpu.VMEM((BPT,), jnp.float32),
            pltpu.SemaphoreType.DMA,
            pltpu.SemaphoreType.DMA,
        ],
    )
    def gather_kernel(tableT_hbm, fmseg_hbm, inputsT_hbm, fidx_hbm,
                      embT_out, fm_out,
                      plane_v, idx_v, out_v, fidx_v, fm1_v, sem_p, sem_o):
        wid = lax.axis_index("s") * NC + lax.axis_index("c")
        p0 = wid * PPT
        f0 = p0 // D
        f1 = (p0 + PPT - 1) // D

        pltpu.sync_copy(inputsT_hbm.at[f0], idx_v.at[0])
        pltpu.sync_copy(inputsT_hbm.at[f1], idx_v.at[1])

        out_cp = [None, None]
        for i in range(PPT):
            p = p0 + i
            f = p // D
            d = p % D
            pltpu.async_copy(tableT_hbm.at[f, d], plane_v, sem_p).wait()
            if out_cp[i % 2] is not None:
                out_cp[i % 2].wait()
            sel = jnp.where(f == f0, 0, 1)

            def body(j, _):
                for u in range(4):
                    jj = j * 4 + u
                    idx16 = idx_v[sel, pl.ds(jj * 16, 16)]
                    out_v[i % 2, pl.ds(jj * 16, 16)] = plsc.load_gather(
                        plane_v, [idx16])
                return ()

            lax.fori_loop(0, B // 64, body, ())
            out_cp[i % 2] = pltpu.async_copy(
                out_v.at[i % 2], embT_out.at[p], sem_o)
        for c in out_cp:
            if c is not None:
                c.wait()

        pltpu.sync_copy(fmseg_hbm, plane_v)
        pltpu.sync_copy(fidx_hbm.at[wid], fidx_v)

        def fm_body(j, _):
            acc = jnp.zeros((16,), jnp.float32)
            for f in range(F):
                idx16 = fidx_v[f, pl.ds(j * 16, 16)]
                acc = acc + plsc.load_gather(plane_v, [idx16])
            fm1_v[pl.ds(j * 16, 16)] = acc
            return ()

        lax.fori_loop(0, BPT // 16, fm_body, ())
        pltpu.sync_copy(fm1_v, fm_out.at[pl.ds(wid * BPT, BPT)])

    return gather_kernel(tableT, fmseg, inputsT, fidx3)


def _mlp_body(x_ref, fm_ref, w0_ref, b0_ref, w1_ref, b1_ref, w2_ref, b2_ref,
              dw_ref, db_ref, out_ref):
    x = x_ref[...]
    first = fm_ref[...]
    row = lax.broadcasted_iota(jnp.int32, (FD, D), 0)
    col = lax.broadcasted_iota(jnp.int32, (FD, D), 1)
    m = (row % D == col).astype(jnp.float32)
    cdim = (((0,), (0,)), ((), ()))
    s = lax.dot_general(m, x, cdim, preferred_element_type=jnp.float32)
    second = 0.5 * (jnp.sum(s * s, axis=0, keepdims=True)
                    - jnp.sum(x * x, axis=0, keepdims=True))
    h = jnp.maximum(
        lax.dot_general(w0_ref[...], x, cdim,
                        preferred_element_type=jnp.float32) + b0_ref[...], 0.0)
    h = jnp.maximum(
        lax.dot_general(w1_ref[...], h, cdim,
                        preferred_element_type=jnp.float32) + b1_ref[...], 0.0)
    h = jnp.maximum(
        lax.dot_general(w2_ref[...], h, cdim,
                        preferred_element_type=jnp.float32) + b2_ref[...], 0.0)
    deep = lax.dot_general(dw_ref[...], h, cdim,
                           preferred_element_type=jnp.float32) + db_ref[...]
    out_ref[...] = jax.nn.sigmoid(first + second + deep)


def _tc_mlp(embT, fm_first, W0, b0, W1, b1, W2, b2, dense_W, dense_b):
    BB = 1024
    grid = (B // BB,)
    const = lambda shape: pl.BlockSpec(shape, lambda i: (0, 0))
    return pl.pallas_call(
        _mlp_body,
        grid=grid,
        in_specs=[
            pl.BlockSpec((FD, BB), lambda i: (0, i)),
            pl.BlockSpec((1, BB), lambda i: (0, i)),
            const((FD, HID[0])),
            const((HID[0], 1)),
            const((HID[0], HID[1])),
            const((HID[1], 1)),
            const((HID[1], HID[2])),
            const((HID[2], 1)),
            const((HID[2], 1)),
            const((1, 1)),
        ],
        out_specs=pl.BlockSpec((1, BB), lambda i: (0, i)),
        out_shape=jax.ShapeDtypeStruct((1, B), jnp.float32),
    )(embT, fm_first.reshape(1, B), W0, b0.reshape(-1, 1), W1,
      b1.reshape(-1, 1), W2, b2.reshape(-1, 1), dense_W,
      dense_b.reshape(1, 1))


def kernel(inputs, emb_tables, fm_w, W0, b0, W1, b1, W2, b2, dense_W, dense_b):
    tableT = emb_tables.transpose(0, 2, 1)
    fmseg = fm_w[V:2 * V, 0]
    inputsT = inputs.T
    fidx3 = inputsT.reshape(F, NW, BPT).transpose(1, 0, 2)
    embT, fm_first = _sc_gather(tableT, fmseg, inputsT, fidx3)
    out = _tc_mlp(embT, fm_first, W0, b0, W1, b1, W2, b2, dense_W, dense_b)
    return out.reshape(B, 1)


# speedup vs baseline: 1.0814x; 1.0065x over previous
import functools

import jax
import jax.numpy as jnp
from jax import lax
from jax.experimental import pallas as pl
from jax.experimental.pallas import tpu as pltpu
from jax.experimental.pallas import tpu_sc as plsc

B, F, V, D = 4096, 26, 100000, 16
FD = F * D
HID = (256, 128, 64)

NC, NS = 2, 16
NW = NC * NS
PPT = FD // NW
BPT = B // NW


def _sc_gather(tableT, fmseg, inputsT, fidx3):
    mesh = plsc.VectorSubcoreMesh(core_axis_name="c", subcore_axis_name="s")

    @functools.partial(
        pl.kernel,
        mesh=mesh,
        compiler_params=pltpu.CompilerParams(use_tc_tiling_on_sc=True,
                                             needs_layout_passes=False),
        out_type=[
            jax.ShapeDtypeStruct((FD, B), jnp.float32),
            jax.ShapeDtypeStruct((B,), jnp.float32),
        ],
        scratch_types=[
            pltpu.VMEM((V,), jnp.float32),
            pltpu.VMEM((2, B), jnp.int32),
            pltpu.VMEM((2, B), jnp.float32),
            pltpu.VMEM((F, BPT), jnp.int32),
            pltpu.VMEM((BPT,), jnp.float32),
            pltpu.SemaphoreType.DMA,
            pltpu.SemaphoreType.DMA,
        ],
    )
    def gather_kernel(tableT_hbm, fmseg_hbm, inputsT_hbm, fidx_hbm,
                      embT_out, fm_out,
                      plane_v, idx_v, out_v, fidx_v, fm1_v, sem_p, sem_o):
        wid = lax.axis_index("s") * NC + lax.axis_index("c")
        p0 = wid * PPT
        f0 = p0 // D
        f1 = (p0 + PPT - 1) // D

        pltpu.sync_copy(inputsT_hbm.at[f0], idx_v.at[0])
        pltpu.sync_copy(inputsT_hbm.at[f1], idx_v.at[1])

        out_cp = [None, None]
        for i in range(PPT):
            p = p0 + i
            f = p // D
            d = p % D
            pltpu.async_copy(tableT_hbm.at[f, d], plane_v, sem_p).wait()
            if out_cp[i % 2] is not None:
                out_cp[i % 2].wait()
            sel = jnp.where(f == f0, 0, 1)

            def body(j, _):
                for u in range(4):
                    jj = j * 4 + u
                    idx16 = idx_v[sel, pl.ds(jj * 16, 16)]
                    out_v[i % 2, pl.ds(jj * 16, 16)] = plsc.load_gather(
                        plane_v, [idx16])
                return ()

            lax.fori_loop(0, B // 64, body, ())
            out_cp[i % 2] = pltpu.async_copy(
                out_v.at[i % 2], embT_out.at[p], sem_o)
        for c in out_cp:
            if c is not None:
                c.wait()

        pltpu.sync_copy(fmseg_hbm, plane_v)
        pltpu.sync_copy(fidx_hbm.at[wid], fidx_v)

        def fm_body(j, _):
            acc = jnp.zeros((16,), jnp.float32)
            for f in range(F):
                idx16 = fidx_v[f, pl.ds(j * 16, 16)]
                acc = acc + plsc.load_gather(plane_v, [idx16])
            fm1_v[pl.ds(j * 16, 16)] = acc
            return ()

        lax.fori_loop(0, BPT // 16, fm_body, ())
        pltpu.sync_copy(fm1_v, fm_out.at[pl.ds(wid * BPT, BPT)])

    return gather_kernel(tableT, fmseg, inputsT, fidx3)


def _mlp_body(x_ref, fm_ref, w0_ref, b0_ref, w1_ref, b1_ref, w2_ref, b2_ref,
              dw_ref, db_ref, out_ref):
    x = x_ref[...]
    first = fm_ref[...]
    row = lax.broadcasted_iota(jnp.int32, (FD, D), 0)
    col = lax.broadcasted_iota(jnp.int32, (FD, D), 1)
    m = (row % D == col).astype(jnp.float32)
    cdim = (((0,), (0,)), ((), ()))
    s = lax.dot_general(m, x, cdim, preferred_element_type=jnp.float32)
    second = 0.5 * (jnp.sum(s * s, axis=0, keepdims=True)
                    - jnp.sum(x * x, axis=0, keepdims=True))
    h = jnp.maximum(
        lax.dot_general(w0_ref[...], x, cdim,
                        preferred_element_type=jnp.float32) + b0_ref[...], 0.0)
    h = jnp.maximum(
        lax.dot_general(w1_ref[...], h, cdim,
                        preferred_element_type=jnp.float32) + b1_ref[...], 0.0)
    h = jnp.maximum(
        lax.dot_general(w2_ref[...], h, cdim,
                        preferred_element_type=jnp.float32) + b2_ref[...], 0.0)
    deep = lax.dot_general(dw_ref[...], h, cdim,
                           preferred_element_type=jnp.float32) + db_ref[...]
    out_ref[...] = jax.nn.sigmoid(first + second + deep)


def _tc_mlp(embT, fm_first, W0, b0, W1, b1, W2, b2, dense_W, dense_b):
    BB = 2048
    grid = (B // BB,)
    const = lambda shape: pl.BlockSpec(shape, lambda i: (0, 0))
    return pl.pallas_call(
        _mlp_body,
        grid=grid,
        in_specs=[
            pl.BlockSpec((FD, BB), lambda i: (0, i)),
            pl.BlockSpec((1, BB), lambda i: (0, i)),
            const((FD, HID[0])),
            const((HID[0], 1)),
            const((HID[0], HID[1])),
            const((HID[1], 1)),
            const((HID[1], HID[2])),
            const((HID[2], 1)),
            const((HID[2], 1)),
            const((1, 1)),
        ],
        out_specs=pl.BlockSpec((1, BB), lambda i: (0, i)),
        out_shape=jax.ShapeDtypeStruct((1, B), jnp.float32),
    )(embT, fm_first.reshape(1, B), W0, b0.reshape(-1, 1), W1,
      b1.reshape(-1, 1), W2, b2.reshape(-1, 1), dense_W,
      dense_b.reshape(1, 1))


def kernel(inputs, emb_tables, fm_w, W0, b0, W1, b1, W2, b2, dense_W, dense_b):
    tableT = emb_tables.transpose(0, 2, 1)
    fmseg = fm_w[V:2 * V, 0]
    inputsT = inputs.T
    fidx3 = inputsT.reshape(F, NW, BPT).transpose(1, 0, 2)
    embT, fm_first = _sc_gather(tableT, fmseg, inputsT, fidx3)
    out = _tc_mlp(embT, fm_first, W0, b0, W1, b1, W2, b2, dense_W, dense_b)
    return out.reshape(B, 1)


# speedup vs baseline: 1.0822x; 1.0008x over previous
import functools

import jax
import jax.numpy as jnp
from jax import lax
from jax.experimental import pallas as pl
from jax.experimental.pallas import tpu as pltpu
from jax.experimental.pallas import tpu_sc as plsc

B, F, V, D = 4096, 26, 100000, 16
FD = F * D
HID = (256, 128, 64)

NC, NS = 2, 16
NW = NC * NS
PPT = FD // NW
BPT = B // NW


def _sc_gather(tableT, fmseg, inputsT, fidx3):
    mesh = plsc.VectorSubcoreMesh(core_axis_name="c", subcore_axis_name="s")

    @functools.partial(
        pl.kernel,
        mesh=mesh,
        compiler_params=pltpu.CompilerParams(use_tc_tiling_on_sc=True,
                                             needs_layout_passes=False),
        out_type=[
            jax.ShapeDtypeStruct((FD, B), jnp.float32),
            jax.ShapeDtypeStruct((B,), jnp.float32),
        ],
        scratch_types=[
            pltpu.VMEM((V,), jnp.float32),
            pltpu.VMEM((2, B), jnp.int32),
            pltpu.VMEM((2, B), jnp.float32),
            pltpu.VMEM((F, BPT), jnp.int32),
            pltpu.VMEM((BPT,), jnp.float32),
            pltpu.SemaphoreType.DMA,
            pltpu.SemaphoreType.DMA,
        ],
    )
    def gather_kernel(tableT_hbm, fmseg_hbm, inputsT_hbm, fidx_hbm,
                      embT_out, fm_out,
                      plane_v, idx_v, out_v, fidx_v, fm1_v, sem_p, sem_o):
        wid = lax.axis_index("s") * NC + lax.axis_index("c")
        p0 = wid * PPT
        f0 = p0 // D
        f1 = (p0 + PPT - 1) // D

        pltpu.sync_copy(inputsT_hbm.at[f0], idx_v.at[0])
        pltpu.sync_copy(inputsT_hbm.at[f1], idx_v.at[1])

        out_cp = [None, None]
        for i in range(PPT):
            p = p0 + i
            f = p // D
            d = p % D
            pltpu.async_copy(tableT_hbm.at[f, d], plane_v, sem_p).wait()
            if out_cp[i % 2] is not None:
                out_cp[i % 2].wait()
            sel = jnp.where(f == f0, 0, 1)

            def body(j, _):
                for u in range(4):
                    jj = j * 4 + u
                    idx16 = idx_v[sel, pl.ds(jj * 16, 16)]
                    out_v[i % 2, pl.ds(jj * 16, 16)] = plsc.load_gather(
                        plane_v, [idx16])
                return ()

            lax.fori_loop(0, B // 64, body, ())
            out_cp[i % 2] = pltpu.async_copy(
                out_v.at[i % 2], embT_out.at[p], sem_o)
        for c in out_cp:
            if c is not None:
                c.wait()

        pltpu.sync_copy(fmseg_hbm, plane_v)
        pltpu.sync_copy(fidx_hbm.at[wid], fidx_v)

        def fm_body(j, _):
            acc = jnp.zeros((16,), jnp.float32)
            for f in range(F):
                idx16 = fidx_v[f, pl.ds(j * 16, 16)]
                acc = acc + plsc.load_gather(plane_v, [idx16])
            fm1_v[pl.ds(j * 16, 16)] = acc
            return ()

        lax.fori_loop(0, BPT // 16, fm_body, ())
        pltpu.sync_copy(fm1_v, fm_out.at[pl.ds(wid * BPT, BPT)])

    return gather_kernel(tableT, fmseg, inputsT, fidx3)


def _mlp_body(x_ref, fm_ref, w0_ref, b0_ref, w1_ref, b1_ref, w2_ref, b2_ref,
              dw_ref, db_ref, out_ref):
    x = x_ref[...]
    first = fm_ref[...]
    row = lax.broadcasted_iota(jnp.int32, (FD, D), 0)
    col = lax.broadcasted_iota(jnp.int32, (FD, D), 1)
    m = (row % D == col).astype(jnp.float32)
    cdim = (((0,), (0,)), ((), ()))
    s = lax.dot_general(m, x, cdim, preferred_element_type=jnp.float32)
    second = 0.5 * (jnp.sum(s * s, axis=0, keepdims=True)
                    - jnp.sum(x * x, axis=0, keepdims=True))
    h = jnp.maximum(
        lax.dot_general(w0_ref[...], x, cdim,
                        preferred_element_type=jnp.float32) + b0_ref[...], 0.0)
    h = jnp.maximum(
        lax.dot_general(w1_ref[...], h, cdim,
                        preferred_element_type=jnp.float32) + b1_ref[...], 0.0)
    h = jnp.maximum(
        lax.dot_general(w2_ref[...], h, cdim,
                        preferred_element_type=jnp.float32) + b2_ref[...], 0.0)
    deep = lax.dot_general(dw_ref[...], h, cdim,
                           preferred_element_type=jnp.float32) + db_ref[...]
    out_ref[...] = jax.nn.sigmoid(first + second + deep)


def _tc_mlp(embT, fm_first, W0, b0, W1, b1, W2, b2, dense_W, dense_b):
    BB = 4096
    grid = (B // BB,)
    const = lambda shape: pl.BlockSpec(shape, lambda i: (0, 0))
    return pl.pallas_call(
        _mlp_body,
        grid=grid,
        in_specs=[
            pl.BlockSpec((FD, BB), lambda i: (0, i)),
            pl.BlockSpec((1, BB), lambda i: (0, i)),
            const((FD, HID[0])),
            const((HID[0], 1)),
            const((HID[0], HID[1])),
            const((HID[1], 1)),
            const((HID[1], HID[2])),
            const((HID[2], 1)),
            const((HID[2], 1)),
            const((1, 1)),
        ],
        out_specs=pl.BlockSpec((1, BB), lambda i: (0, i)),
        out_shape=jax.ShapeDtypeStruct((1, B), jnp.float32),
    )(embT, fm_first.reshape(1, B), W0, b0.reshape(-1, 1), W1,
      b1.reshape(-1, 1), W2, b2.reshape(-1, 1), dense_W,
      dense_b.reshape(1, 1))


def kernel(inputs, emb_tables, fm_w, W0, b0, W1, b1, W2, b2, dense_W, dense_b):
    tableT = emb_tables.transpose(0, 2, 1)
    fmseg = fm_w[V:2 * V, 0]
    inputsT = inputs.T
    fidx3 = inputsT.reshape(F, NW, BPT).transpose(1, 0, 2)
    embT, fm_first = _sc_gather(tableT, fmseg, inputsT, fidx3)
    out = _tc_mlp(embT, fm_first, W0, b0, W1, b1, W2, b2, dense_W, dense_b)
    return out.reshape(B, 1)


# speedup vs baseline: 1.0837x; 1.0014x over previous
import functools

import jax
import jax.numpy as jnp
from jax import lax
from jax.experimental import pallas as pl
from jax.experimental.pallas import tpu as pltpu
from jax.experimental.pallas import tpu_sc as plsc

B, F, V, D = 4096, 26, 100000, 16
FD = F * D
HID = (256, 128, 64)

NC, NS = 2, 16
NW = NC * NS
PPT = FD // NW
BPT = B // NW


def _sc_gather(tableT, fmseg, inputsT, fidx3):
    mesh = plsc.VectorSubcoreMesh(core_axis_name="c", subcore_axis_name="s")

    @functools.partial(
        pl.kernel,
        mesh=mesh,
        compiler_params=pltpu.CompilerParams(use_tc_tiling_on_sc=True,
                                             needs_layout_passes=False),
        out_type=[
            jax.ShapeDtypeStruct((FD, B), jnp.float32),
            jax.ShapeDtypeStruct((B,), jnp.float32),
        ],
        scratch_types=[
            pltpu.VMEM((V,), jnp.float32),
            pltpu.VMEM((2, B), jnp.int32),
            pltpu.VMEM((2, B), jnp.float32),
            pltpu.VMEM((F, BPT), jnp.int32),
            pltpu.VMEM((BPT,), jnp.float32),
            pltpu.SemaphoreType.DMA,
            pltpu.SemaphoreType.DMA,
        ],
    )
    def gather_kernel(tableT_hbm, fmseg_hbm, inputsT_hbm, fidx_hbm,
                      embT_out, fm_out,
                      plane_v, idx_v, out_v, fidx_v, fm1_v, sem_p, sem_o):
        wid = lax.axis_index("s") * NC + lax.axis_index("c")
        p0 = wid * PPT
        f0 = p0 // D
        f1 = (p0 + PPT - 1) // D

        pltpu.sync_copy(inputsT_hbm.at[f0], idx_v.at[0])
        pltpu.sync_copy(inputsT_hbm.at[f1], idx_v.at[1])

        out_cp = [None, None]
        for i in range(PPT):
            p = p0 + i
            f = p // D
            d = p % D
            pltpu.async_copy(tableT_hbm.at[f, d], plane_v, sem_p).wait()
            if out_cp[i % 2] is not None:
                out_cp[i % 2].wait()
            sel = jnp.where(f == f0, 0, 1)

            def body(j, _):
                for u in range(4):
                    jj = j * 4 + u
                    idx16 = idx_v[sel, pl.ds(jj * 16, 16)]
                    out_v[i % 2, pl.ds(jj * 16, 16)] = plsc.load_gather(
                        plane_v, [idx16])
                return ()

            lax.fori_loop(0, B // 64, body, ())
            out_cp[i % 2] = pltpu.async_copy(
                out_v.at[i % 2], embT_out.at[p], sem_o)
        for c in out_cp:
            if c is not None:
                c.wait()

        pltpu.sync_copy(fmseg_hbm, plane_v)
        pltpu.sync_copy(fidx_hbm.at[wid], fidx_v)

        def fm_body(j, _):
            acc = jnp.zeros((16,), jnp.float32)
            for f in range(F):
                idx16 = fidx_v[f, pl.ds(j * 16, 16)]
                acc = acc + plsc.load_gather(plane_v, [idx16])
            fm1_v[pl.ds(j * 16, 16)] = acc
            return ()

        lax.fori_loop(0, BPT // 16, fm_body, ())
        pltpu.sync_copy(fm1_v, fm_out.at[pl.ds(wid * BPT, BPT)])

    return gather_kernel(tableT, fmseg, inputsT, fidx3)


def _mlp_body(x_ref, fm_ref, w0_ref, b0_ref, w1_ref, b1_ref, w2_ref, b2_ref,
              dw_ref, db_ref, out_ref):
    x = x_ref[...]
    first = fm_ref[...]
    row = lax.broadcasted_iota(jnp.int32, (FD, D), 0)
    col = lax.broadcasted_iota(jnp.int32, (FD, D), 1)
    m = (row % D == col).astype(jnp.float32)
    cdim = (((0,), (0,)), ((), ()))
    s = lax.dot_general(m, x, cdim, preferred_element_type=jnp.float32)
    second = 0.5 * (jnp.sum(s * s, axis=0, keepdims=True)
                    - jnp.sum(x * x, axis=0, keepdims=True))
    h = jnp.maximum(
        lax.dot_general(w0_ref[...], x, cdim,
                        preferred_element_type=jnp.float32) + b0_ref[...], 0.0)
    h = jnp.maximum(
        lax.dot_general(w1_ref[...], h, cdim,
                        preferred_element_type=jnp.float32) + b1_ref[...], 0.0)
    h = jnp.maximum(
        lax.dot_general(w2_ref[...], h, cdim,
                        preferred_element_type=jnp.float32) + b2_ref[...], 0.0)
    deep = lax.dot_general(dw_ref[...], h, cdim,
                           preferred_element_type=jnp.float32) + db_ref[...]
    out_ref[...] = jax.nn.sigmoid(first + second + deep)


def _tc_mlp(embT, fm_first, W0, b0, W1, b1, W2, b2, dense_W, dense_b):
    BB = 2048
    grid = (B // BB,)
    const = lambda shape: pl.BlockSpec(shape, lambda i: (0, 0))
    return pl.pallas_call(
        _mlp_body,
        grid=grid,
        in_specs=[
            pl.BlockSpec((FD, BB), lambda i: (0, i)),
            pl.BlockSpec((1, BB), lambda i: (0, i)),
            const((FD, HID[0])),
            const((HID[0], 1)),
            const((HID[0], HID[1])),
            const((HID[1], 1)),
            const((HID[1], HID[2])),
            const((HID[2], 1)),
            const((HID[2], 1)),
            const((1, 1)),
        ],
        out_specs=pl.BlockSpec((1, BB), lambda i: (0, i)),
        out_shape=jax.ShapeDtypeStruct((1, B), jnp.float32),
    )(embT, fm_first.reshape(1, B), W0, b0.reshape(-1, 1), W1,
      b1.reshape(-1, 1), W2, b2.reshape(-1, 1), dense_W,
      dense_b.reshape(1, 1))


def kernel(inputs, emb_tables, fm_w, W0, b0, W1, b1, W2, b2, dense_W, dense_b):
    tableT = emb_tables.transpose(0, 2, 1)
    fmseg = fm_w[V:2 * V, 0]
    inputsT = inputs.T
    fidx3 = inputsT.reshape(F, NW, BPT).transpose(1, 0, 2)
    embT, fm_first = _sc_gather(tableT, fmseg, inputsT, fidx3)
    out = _tc_mlp(embT, fm_first, W0, b0, W1, b1, W2, b2, dense_W, dense_b)
    return out.reshape(B, 1)
